# baseline (speedup 1.0000x reference)
import functools

import jax
import jax.numpy as jnp
from jax import lax
from jax.experimental import pallas as pl
from jax.experimental.pallas import tpu as pltpu
from jax.experimental.pallas import tpu_sc as plsc

VOCAB = 1000000
EMB = 32
BATCH = 16384
HIST = 50

NC, NS = 2, 16
NW = NC * NS
COLS = BATCH // NW
BT = BATCH // 128
TPW = BT // NW
HALF = 256
NBUF = 2
L = 16

_mesh = plsc.VectorSubcoreMesh(core_axis_name="c", subcore_axis_name="s")


@functools.partial(
    pl.kernel,
    out_type=jax.ShapeDtypeStruct((HIST, BATCH // 4, 128), jnp.float32),
    mesh=_mesh,
    scratch_types=[
        pltpu.VMEM((NBUF, 2, 128), jnp.int32),
        pltpu.VMEM((NBUF, 2, 128), jnp.int32),
        pltpu.VMEM((NBUF, 2, 128, 128), jnp.float32),
        pltpu.VMEM((NBUF, 64, 128), jnp.float32),
        [pltpu.SemaphoreType.DMA] * NBUF,
        [pltpu.SemaphoreType.DMA] * NBUF,
    ],
    compiler_params=pltpu.CompilerParams(use_tc_tiling_on_sc=True),
)
def _gather_kernel(xt_hbm, table_hbm, out_hbm, idx_v, grp_v,
                   rows_v, ext_v, gsem, wsem):
    wid = lax.axis_index("s") * NC + lax.axis_index("c")
    t0 = wid * TPW
    g0 = wid * (COLS // 4)

    def start_chunk(h, b):
        pltpu.sync_copy(xt_hbm.at[h, pl.ds(t0 + 2 * b, 2), :], idx_v.at[b])
        for r in range(2):
            for k in range(128 // L):
                v = idx_v[b, r, pl.ds(k * L, L)]
                grp_v[b, r, pl.ds(k * L, L)] = v >> 2
        gathers = []
        for r in range(2):
            gathers.append(pltpu.async_copy(
                table_hbm.at[grp_v.at[b, r]], rows_v.at[b, r], gsem[b]))
        return gathers

    def extract(b):
        for r in range(2):
            src = rows_v.at[b, r]
            for k in range(128 // L):
                v16 = idx_v[b, r, pl.ds(k * L, L)]
                o16 = (v16 & 3) << 5
                for j in range(L):
                    i = k * L + j
                    o = o16[j]
                    grow = r * 32 + i // 4
                    gc0 = (i % 4) * 32
                    ext_v[b, grow, pl.ds(gc0, L)] = src[i, pl.ds(o, L)]
                    ext_v[b, grow, pl.ds(gc0 + L, L)] = src[i, pl.ds(o + L, L)]

    def body(h, _):
        gathers = [start_chunk(h, b) for b in range(NBUF)]
        for b in range(NBUF):
            for g in gathers[b]:
                g.wait()
            extract(b)
            pltpu.async_copy(
                ext_v.at[b], out_hbm.at[h, pl.ds(g0 + b * 64, 64), :],
                wsem[b])
            if True:
                pass
        return 0

    def body_wrapped(h, carry):
        @pl.when(h > 0)
        def _():
            for b in range(NBUF):
                pltpu.make_async_copy(
                    ext_v.at[b], out_hbm.at[0, pl.ds(0, 64), :],
                    wsem[b]).wait()
        body(h, carry)
        return 0

    lax.fori_loop(0, HIST, body_wrapped, 0)
    for b in range(NBUF):
        pltpu.make_async_copy(
            ext_v.at[b], out_hbm.at[0, pl.ds(0, 64), :], wsem[b]).wait()


def kernel(x, table):
    xt = x.T.astype(jnp.int32).reshape(HIST, BATCH // 128, 128)
    tr = table.reshape(VOCAB // 4, 128)
    out = _gather_kernel(xt, tr)
    return out.reshape(HIST, BATCH, EMB).transpose(1, 0, 2)
